# Initial kernel scaffold; baseline (speedup 1.0000x reference)
import functools

import jax
import jax.numpy as jnp
from jax import lax
from jax.experimental import pallas as pl
from jax.experimental.pallas import tpu as pltpu
from jax.experimental.pallas import tpu_sc as plsc

VOCAB = 1_000_000
D = 16
B = 16384
L = 200

NC = 2
NS = 16
NW = NC * NS

RPW = B // NW
CH_ROWS = 64
NCH = RPW // CH_ROWS
CH_IDX = CH_ROWS * L
GW = 128
NSTEP = CH_IDX // GW


_A_ROWS = 125_000
_A_BLK = 5_000
_A_GRID = _A_ROWS // _A_BLK


def _tv_body(t_ref, w_ref, b_ref, o_ref):
    acc = jnp.dot(t_ref[...], w_ref[...], preferred_element_type=jnp.float32)
    o_ref[...] = (acc + b_ref[0, 0]) * (1.0 / L)


def _compute_tv(table, W, b):
    w = W[:, 0]
    wb = (jnp.eye(8, dtype=jnp.float32)[:, None, :] * w[None, :, None])
    wb = wb.reshape(128, 8)
    tv2 = pl.pallas_call(
        _tv_body,
        grid=(_A_GRID,),
        in_specs=[
            pl.BlockSpec((_A_BLK, 128), lambda i: (i, 0)),
            pl.BlockSpec((128, 8), lambda i: (0, 0)),
            pl.BlockSpec(memory_space=pltpu.SMEM),
        ],
        out_specs=pl.BlockSpec((_A_BLK, 8), lambda i: (i, 0)),
        out_shape=jax.ShapeDtypeStruct((_A_ROWS, 8), jnp.float32),
    )(table.reshape(_A_ROWS, 128), wb, b.reshape(1, 1))
    return tv2.reshape(VOCAB)


_mesh = plsc.VectorSubcoreMesh(
    core_axis_name="c", subcore_axis_name="s", num_cores=NC, num_subcores=NS)


@functools.partial(
    pl.kernel,
    out_type=jax.ShapeDtypeStruct((B // 16, 16), jnp.float32),
    mesh=_mesh,
    scratch_types=[
        pltpu.VMEM_SHARED((VOCAB,), jnp.float32),
        pltpu.VMEM((NSTEP, GW), jnp.int32),
        pltpu.VMEM((NSTEP, GW), jnp.float32),
        pltpu.VMEM((RPW // 16, 16), jnp.float32),
        pltpu.SemaphoreType.DMA,
    ],
)
def _sc_pool(tv_hbm, x_hbm, out_hbm, tv_sp, idx_v, vals_v, out_v, sem):
    c = lax.axis_index("c")
    s = lax.axis_index("s")
    wid = s * NC + c

    qt = VOCAB // 4

    @pl.when(s < 4)
    def _():
        pltpu.sync_copy(tv_hbm.at[pl.ds(s * qt, qt)], tv_sp.at[pl.ds(s * qt, qt)])

    plsc.subcore_barrier()

    lane = lax.iota(jnp.int32, 16)
    xrow0 = wid * (RPW * L // GW)

    def chunk_body(ch, carry):
        pltpu.sync_copy(x_hbm.at[pl.ds(xrow0 + ch * NSTEP, NSTEP), :], idx_v)

        def g_issue(t, cr):
            pltpu.async_copy(tv_sp.at[idx_v.at[t]], vals_v.at[t], sem)
            return cr

        lax.fori_loop(0, NSTEP, g_issue, 0)

        def g_drain(t, cr):
            pltpu.make_async_copy(
                tv_hbm.at[pl.ds(0, GW)], vals_v.at[t], sem).wait()
            return cr

        lax.fori_loop(0, NSTEP, g_drain, 0)

        def red_g(g, cr):
            row_off = (g * 16 + lane) * L

            def red_j(j, acc):
                p = row_off + j
                v = plsc.load_gather(
                    vals_v, [lax.shift_right_logical(p, 7),
                             lax.bitwise_and(p, 127)])
                return acc + v

            acc = lax.fori_loop(0, L, red_j, jnp.zeros((16,), jnp.float32))
            sig = 1.0 / (1.0 + jnp.exp(-acc))
            out_v[ch * (CH_ROWS // 16) + g] = sig
            return cr

        lax.fori_loop(0, CH_ROWS // 16, red_g, 0)
        return carry

    lax.fori_loop(0, NCH, chunk_body, 0)

    pltpu.sync_copy(out_v, out_hbm.at[pl.ds(wid * (RPW // 16), RPW // 16), :])


def kernel(x, table, W, b):
    tv = _compute_tv(table, W, b)
    x2 = x.reshape(B * L // GW, GW)
    out = _sc_pool(tv, x2)
    return out.reshape(B, 1)


# speedup vs baseline: 9.1646x; 9.1646x over previous
import functools

import jax
import jax.numpy as jnp
from jax import lax
from jax.experimental import pallas as pl
from jax.experimental.pallas import tpu as pltpu
from jax.experimental.pallas import tpu_sc as plsc

VOCAB = 1_000_000
VOCAB_PAD = 1_000_448
D = 16
B = 16384
L = 200

NC = 2
NS = 16
NW = NC * NS

GPT = (B // 16) // NW
RPG = L * 16 // 128
RPT = GPT * RPG
CHG = 8
NCH = GPT // CHG
CHR = CHG * RPG


_A_ROWS = 125_000
_A_ROWS_PAD = VOCAB_PAD // 8
_A_BLK = 5_000
_A_GRID = _A_ROWS // _A_BLK


def _tv_body(t_ref, w_ref, b_ref, o_ref):
    acc = jnp.dot(t_ref[...], w_ref[...], preferred_element_type=jnp.float32)
    o_ref[...] = (acc + b_ref[0, 0]) * (1.0 / L)


def _compute_tv(table, W, b):
    w = W[:, 0]
    wb = (jnp.eye(8, dtype=jnp.float32)[:, None, :] * w[None, :, None])
    wb = wb.reshape(128, 8)
    tv2 = pl.pallas_call(
        _tv_body,
        grid=(_A_GRID,),
        in_specs=[
            pl.BlockSpec((_A_BLK, 128), lambda i: (i, 0)),
            pl.BlockSpec((128, 8), lambda i: (0, 0)),
            pl.BlockSpec(memory_space=pltpu.SMEM),
        ],
        out_specs=pl.BlockSpec((_A_BLK, 8), lambda i: (i, 0)),
        out_shape=jax.ShapeDtypeStruct((_A_ROWS_PAD, 8), jnp.float32),
    )(table.reshape(_A_ROWS, 128), wb, b.reshape(1, 1))
    return tv2.reshape(VOCAB_PAD)


_T_BLK = 256
_T_GRID = B // _T_BLK


def _xt_body(x_ref, o_ref):
    a = x_ref[...]
    o_ref[...] = jnp.swapaxes(a.reshape(16, 16, L), 1, 2)


def _transpose_x(x):
    xt3 = pl.pallas_call(
        _xt_body,
        grid=(_T_GRID,),
        in_specs=[pl.BlockSpec((_T_BLK, L), lambda i: (i, 0))],
        out_specs=pl.BlockSpec((16, L, 16), lambda i: (i, 0, 0)),
        out_shape=jax.ShapeDtypeStruct((B // 16, L, 16), jnp.int32),
    )(x)
    return xt3.reshape(B * L // 128, 128)


_mesh = plsc.VectorSubcoreMesh(
    core_axis_name="c", subcore_axis_name="s", num_cores=NC, num_subcores=NS)


@functools.partial(
    pl.kernel,
    out_type=jax.ShapeDtypeStruct((B // 16, 16), jnp.float32),
    mesh=_mesh,
    scratch_types=[
        pltpu.VMEM_SHARED((VOCAB_PAD,), jnp.float32),
        pltpu.VMEM((CHR, 128), jnp.int32),
        pltpu.VMEM((CHR * 128,), jnp.float32),
        pltpu.VMEM((GPT, 16), jnp.float32),
        pltpu.SemaphoreType.DMA,
    ],
)
def _sc_pool(tv_hbm, xt_hbm, out_hbm, tv_sp, idx_v, vals_v, out_v, sem):
    c = lax.axis_index("c")
    s = lax.axis_index("s")
    wid = s * NC + c

    off0 = s * (VOCAB_PAD // 16)
    for off, n in ((0, CHR * 128), (CHR * 128, CHR * 128),
                   (2 * CHR * 128, VOCAB_PAD // 16 - 2 * CHR * 128)):
        pltpu.sync_copy(tv_hbm.at[pl.ds(off0 + off, n)], vals_v.at[pl.ds(0, n)])
        pltpu.sync_copy(vals_v.at[pl.ds(0, n)], tv_sp.at[pl.ds(off0 + off, n)])

    plsc.subcore_barrier()

    def chunk_body(ch, carry):
        row0 = wid * RPT + ch * CHR
        pltpu.sync_copy(xt_hbm.at[pl.ds(row0, CHR), :], idx_v)

        def g_issue(t, cr):
            pltpu.async_copy(
                tv_sp.at[idx_v.at[t]], vals_v.at[pl.ds(t * 128, 128)], sem)
            return cr

        lax.fori_loop(0, CHR, g_issue, 0)
        pltpu.make_async_copy(
            tv_hbm.at[pl.ds(0, CHR * 128)], vals_v, sem).wait()

        def red_g(g, cr):
            base = g * RPG * 128

            def red_t(t, acc):
                r = base + t * 128
                for u in range(8):
                    acc = acc + vals_v[pl.ds(r + u * 16, 16)]
                return acc

            acc = lax.fori_loop(0, RPG, red_t, jnp.zeros((16,), jnp.float32))
            sig = 1.0 / (1.0 + jnp.exp(-acc))
            out_v[ch * CHG + g] = sig
            return cr

        lax.fori_loop(0, CHG, red_g, 0)
        return carry

    lax.fori_loop(0, NCH, chunk_body, 0)

    pltpu.sync_copy(out_v, out_hbm.at[pl.ds(wid * GPT, GPT), :])


def kernel(x, table, W, b):
    tv = _compute_tv(table, W, b)
    xt = _transpose_x(x)
    out = _sc_pool(tv, xt)
    return out.reshape(B, 1)
